# Initial kernel scaffold; baseline (speedup 1.0000x reference)
import jax
import jax.numpy as jnp
from jax.experimental import pallas as pl


def kernel(np_batch, table):
    raise NotImplementedError("write your pallas kernel here")


# speedup vs baseline: 4.2049x; 4.2049x over previous
import functools

import jax
import jax.numpy as jnp
from jax import lax
from jax.experimental import pallas as pl
from jax.experimental.pallas import tpu as pltpu
from jax.experimental.pallas import tpu_sc as plsc

BATCH = 1024
SEQ_LEN = 200
WORD_LEN = 20
VOCAB_N = 1000
EMBED_DIM = 32
B_TOTAL = BATCH * SEQ_LEN * WORD_LEN

_INFO = plsc.get_sparse_core_info()
NC = _INFO.num_cores
NS = _INFO.num_subcores
NW = NC * NS
N_PER_W = B_TOTAL // NW

KI = 1024
GROUPS = KI // 16
STEPS = N_PER_W // KI


def _make_emb_kernel():
  mesh = plsc.VectorSubcoreMesh(core_axis_name="c", subcore_axis_name="s")

  @functools.partial(
      pl.kernel,
      mesh=mesh,
      compiler_params=pltpu.CompilerParams(needs_layout_passes=False),
      out_type=jax.ShapeDtypeStruct((B_TOTAL * EMBED_DIM,), jnp.float32),
      scratch_types=[
          pltpu.VMEM((VOCAB_N * EMBED_DIM,), jnp.float32),
          pltpu.VMEM((KI,), jnp.int32),
          pltpu.VMEM((KI * EMBED_DIM,), jnp.float32),
          pltpu.SemaphoreType.DMA,
      ],
  )
  def emb_kernel(idx_hbm, tabf_hbm, out_hbm, tab_v, idx_v, out_v, sem):
    wid = lax.axis_index("s") * NC + lax.axis_index("c")
    pltpu.sync_copy(tabf_hbm, tab_v)

    iota16 = lax.iota(jnp.int32, 16)
    st_iota = iota16 * EMBED_DIM

    def step(t, carry):
      base = wid * N_PER_W + t * KI
      pltpu.sync_copy(idx_hbm.at[pl.ds(base, KI)], idx_v)

      def group(g, carry2):
        iv = idx_v[pl.ds(g * 16, 16)]
        a0 = iv * EMBED_DIM
        sg = g * (16 * EMBED_DIM)
        for c in range(EMBED_DIM):
          val = plsc.load_gather(tab_v, [a0 + c])
          plsc.store_scatter(out_v, [st_iota + (sg + c)], val)
        return carry2

      lax.fori_loop(0, GROUPS, group, 0)
      pltpu.sync_copy(out_v, out_hbm.at[pl.ds(base * EMBED_DIM, KI * EMBED_DIM)])
      return carry

    lax.fori_loop(0, STEPS, step, 0)

  return emb_kernel


_emb = _make_emb_kernel()


@jax.jit
def kernel(np_batch, table):
  flat_idx = np_batch.reshape(-1).astype(jnp.int32)
  flat_tab = table.reshape(-1)
  out = _emb(flat_idx, flat_tab)
  return out.reshape(BATCH, SEQ_LEN, WORD_LEN, EMBED_DIM)


# speedup vs baseline: 5.3037x; 1.2613x over previous
import functools

import jax
import jax.numpy as jnp
from jax import lax
from jax.experimental import pallas as pl
from jax.experimental.pallas import tpu as pltpu
from jax.experimental.pallas import tpu_sc as plsc

BATCH = 1024
SEQ_LEN = 200
WORD_LEN = 20
VOCAB_N = 1000
EMBED_DIM = 32
B_TOTAL = BATCH * SEQ_LEN * WORD_LEN

_INFO = plsc.get_sparse_core_info()
NC = _INFO.num_cores
NS = _INFO.num_subcores
NW = NC * NS
N_PER_W = B_TOTAL // NW

KI = 1280
GROUPS = KI // 16
STEPS = N_PER_W // KI


def _make_emb_kernel():
  mesh = plsc.VectorSubcoreMesh(core_axis_name="c", subcore_axis_name="s")

  @functools.partial(
      pl.kernel,
      mesh=mesh,
      compiler_params=pltpu.CompilerParams(needs_layout_passes=False),
      out_type=jax.ShapeDtypeStruct((B_TOTAL * EMBED_DIM,), jnp.float32),
      scratch_types=[
          pltpu.VMEM((VOCAB_N * EMBED_DIM,), jnp.float32),
          pltpu.VMEM((KI,), jnp.int32),
          pltpu.VMEM((KI,), jnp.int32),
          pltpu.VMEM((KI * EMBED_DIM,), jnp.float32),
          pltpu.VMEM((KI * EMBED_DIM,), jnp.float32),
          pltpu.SemaphoreType.DMA,
          pltpu.SemaphoreType.DMA,
          pltpu.SemaphoreType.DMA,
          pltpu.SemaphoreType.DMA,
          pltpu.SemaphoreType.DMA,
      ],
  )
  def emb_kernel(idx_hbm, tabf_hbm, out_hbm, tab_v, idx_v0, idx_v1,
                 out_v0, out_v1, tab_sem, isem0, isem1, wsem0, wsem1):
    wid = lax.axis_index("s") * NC + lax.axis_index("c")
    base0 = wid * N_PER_W
    pltpu.async_copy(tabf_hbm, tab_v, tab_sem).wait()

    idx_bufs = (idx_v0, idx_v1)
    out_bufs = (out_v0, out_v1)
    isems = (isem0, isem1)
    wsems = (wsem0, wsem1)
    st_iota = lax.iota(jnp.int32, 16) * EMBED_DIM

    def issue_idx(t, b):
      pltpu.async_copy(idx_hbm.at[pl.ds(base0 + t * KI, KI)],
                       idx_bufs[b], isems[b])

    def wait_idx(b):
      pltpu.make_async_copy(idx_hbm.at[pl.ds(0, KI)],
                            idx_bufs[b], isems[b]).wait()

    def issue_wb(t, b):
      pltpu.async_copy(
          out_bufs[b],
          out_hbm.at[pl.ds((base0 + t * KI) * EMBED_DIM, KI * EMBED_DIM)],
          wsems[b])

    def wait_wb(b):
      pltpu.make_async_copy(out_bufs[b],
                            out_hbm.at[pl.ds(0, KI * EMBED_DIM)],
                            wsems[b]).wait()

    def compute(b):
      ivb = idx_bufs[b]
      ovb = out_bufs[b]

      def group(g):
        iv = ivb[pl.ds(g * 16, 16)]
        a0 = iv * EMBED_DIM
        st0 = st_iota + g * (16 * EMBED_DIM)
        for c in range(EMBED_DIM):
          val = plsc.load_gather(tab_v, [a0 + c])
          plsc.store_scatter(ovb, [st0 + c], val)

      plsc.parallel_loop(0, GROUPS, 1, unroll=4)(group)

    issue_idx(0, 0)
    issue_idx(1, 1)

    def step2(tt, carry):
      for b in (0, 1):
        t = 2 * tt + b
        wait_idx(b)

        @pl.when(tt > 0)
        def _():
          wait_wb(b)

        compute(b)
        issue_wb(t, b)

        @pl.when(tt < STEPS // 2 - 1)
        def _():
          issue_idx(t + 2, b)
      return carry

    lax.fori_loop(0, STEPS // 2, step2, 0)
    wait_wb(0)
    wait_wb(1)

  return emb_kernel


_emb = _make_emb_kernel()


@jax.jit
def kernel(np_batch, table):
  flat_idx = np_batch.reshape(-1).astype(jnp.int32)
  flat_tab = table.reshape(-1)
  out = _emb(flat_idx, flat_tab)
  return out.reshape(BATCH, SEQ_LEN, WORD_LEN, EMBED_DIM)


# speedup vs baseline: 5.5410x; 1.0448x over previous
import functools

import jax
import jax.numpy as jnp
from jax import lax
from jax.experimental import pallas as pl
from jax.experimental.pallas import tpu as pltpu
from jax.experimental.pallas import tpu_sc as plsc

BATCH = 1024
SEQ_LEN = 200
WORD_LEN = 20
VOCAB_N = 1000
EMBED_DIM = 32
B_TOTAL = BATCH * SEQ_LEN * WORD_LEN

_INFO = plsc.get_sparse_core_info()
NC = _INFO.num_cores
NS = _INFO.num_subcores
NW = NC * NS
N_PER_W = B_TOTAL // NW

KI = 1280
GROUPS = KI // 16
STEPS = N_PER_W // KI


def _make_emb_kernel():
  mesh = plsc.VectorSubcoreMesh(core_axis_name="c", subcore_axis_name="s")

  @functools.partial(
      pl.kernel,
      mesh=mesh,
      compiler_params=pltpu.CompilerParams(needs_layout_passes=False),
      out_type=jax.ShapeDtypeStruct((B_TOTAL * EMBED_DIM,), jnp.float32),
      scratch_types=[
          pltpu.VMEM((VOCAB_N * EMBED_DIM,), jnp.float32),
          pltpu.VMEM((KI,), jnp.int32),
          pltpu.VMEM((KI,), jnp.int32),
          pltpu.VMEM((KI * EMBED_DIM,), jnp.float32),
          pltpu.VMEM((KI * EMBED_DIM,), jnp.float32),
          pltpu.SemaphoreType.DMA,
          pltpu.SemaphoreType.DMA,
          pltpu.SemaphoreType.DMA,
          pltpu.SemaphoreType.DMA,
          pltpu.SemaphoreType.DMA,
      ],
  )
  def emb_kernel(idx_hbm, tabf_hbm, out_hbm, tab_v, idx_v0, idx_v1,
                 out_v0, out_v1, tab_sem, isem0, isem1, wsem0, wsem1):
    wid = lax.axis_index("s") * NC + lax.axis_index("c")
    base0 = wid * N_PER_W
    pltpu.async_copy(tabf_hbm, tab_v, tab_sem).wait()

    idx_bufs = (idx_v0, idx_v1)
    out_bufs = (out_v0, out_v1)
    isems = (isem0, isem1)
    wsems = (wsem0, wsem1)
    st_iota = lax.iota(jnp.int32, 16) * EMBED_DIM

    def issue_idx(t, b):
      pltpu.async_copy(idx_hbm.at[pl.ds(base0 + t * KI, KI)],
                       idx_bufs[b], isems[b])

    def wait_idx(b):
      pltpu.make_async_copy(idx_hbm.at[pl.ds(0, KI)],
                            idx_bufs[b], isems[b]).wait()

    def issue_wb(t, b):
      pltpu.async_copy(
          out_bufs[b],
          out_hbm.at[pl.ds((base0 + t * KI) * EMBED_DIM, KI * EMBED_DIM)],
          wsems[b])

    def wait_wb(b):
      pltpu.make_async_copy(out_bufs[b],
                            out_hbm.at[pl.ds(0, KI * EMBED_DIM)],
                            wsems[b]).wait()

    def compute(b):
      ivb = idx_bufs[b]
      ovb = out_bufs[b]

      def group(g):
        iv = ivb[pl.ds(g * 16, 16)]
        a0 = iv * EMBED_DIM
        st0 = st_iota + g * (16 * EMBED_DIM)
        vals = [plsc.load_gather(tab_v, [a0 + c]) for c in range(EMBED_DIM)]
        for c in range(EMBED_DIM):
          plsc.store_scatter(ovb, [st0 + c], vals[c])

      plsc.parallel_loop(0, GROUPS, 1, unroll=1)(group)

    issue_idx(0, 0)
    issue_idx(1, 1)

    def step2(tt, carry):
      for b in (0, 1):
        t = 2 * tt + b
        wait_idx(b)

        @pl.when(tt > 0)
        def _():
          wait_wb(b)

        compute(b)
        issue_wb(t, b)

        @pl.when(tt < STEPS // 2 - 1)
        def _():
          issue_idx(t + 2, b)
      return carry

    lax.fori_loop(0, STEPS // 2, step2, 0)
    wait_wb(0)
    wait_wb(1)

  return emb_kernel


_emb = _make_emb_kernel()


@jax.jit
def kernel(np_batch, table):
  flat_idx = np_batch.reshape(-1).astype(jnp.int32)
  flat_tab = table.reshape(-1)
  out = _emb(flat_idx, flat_tab)
  return out.reshape(BATCH, SEQ_LEN, WORD_LEN, EMBED_DIM)


# speedup vs baseline: 7.2800x; 1.3138x over previous
import functools

import jax
import jax.numpy as jnp
from jax import lax
from jax.experimental import pallas as pl
from jax.experimental.pallas import tpu as pltpu
from jax.experimental.pallas import tpu_sc as plsc

BATCH = 1024
SEQ_LEN = 200
WORD_LEN = 20
VOCAB_N = 1000
EMBED_DIM = 32
B_TOTAL = BATCH * SEQ_LEN * WORD_LEN

_INFO = plsc.get_sparse_core_info()
NC = _INFO.num_cores
NS = _INFO.num_subcores
NW = NC * NS
N_PER_W = B_TOTAL // NW

KI = 1280
GROUPS = KI // 16
STEPS = N_PER_W // KI
LOOKAHEAD = 8


def _make_emb_kernel():
  mesh = plsc.VectorSubcoreMesh(core_axis_name="c", subcore_axis_name="s")

  @functools.partial(
      pl.kernel,
      mesh=mesh,
      compiler_params=pltpu.CompilerParams(needs_layout_passes=False),
      out_type=jax.ShapeDtypeStruct((B_TOTAL * EMBED_DIM,), jnp.float32),
      scratch_types=[
          pltpu.VMEM((VOCAB_N * EMBED_DIM,), jnp.float32),
          pltpu.VMEM((KI,), jnp.int32),
          pltpu.VMEM((KI,), jnp.int32),
          pltpu.VMEM((KI * EMBED_DIM,), jnp.float32),
          pltpu.VMEM((KI * EMBED_DIM,), jnp.float32),
          pltpu.SemaphoreType.DMA,
          pltpu.SemaphoreType.DMA,
          pltpu.SemaphoreType.DMA,
          pltpu.SemaphoreType.DMA,
          pltpu.SemaphoreType.DMA,
      ],
  )
  def emb_kernel(idx_hbm, tabf_hbm, out_hbm, tab_v, idx_v0, idx_v1,
                 out_v0, out_v1, tab_sem, isem0, isem1, wsem0, wsem1):
    wid = lax.axis_index("s") * NC + lax.axis_index("c")
    base0 = wid * N_PER_W
    pltpu.async_copy(tabf_hbm, tab_v, tab_sem).wait()

    idx_bufs = (idx_v0, idx_v1)
    out_bufs = (out_v0, out_v1)
    isems = (isem0, isem1)
    wsems = (wsem0, wsem1)
    st_iota = lax.iota(jnp.int32, 16) * EMBED_DIM

    def issue_idx(t, b):
      pltpu.async_copy(idx_hbm.at[pl.ds(base0 + t * KI, KI)],
                       idx_bufs[b], isems[b])

    def wait_idx(b):
      pltpu.make_async_copy(idx_hbm.at[pl.ds(0, KI)],
                            idx_bufs[b], isems[b]).wait()

    def issue_wb(t, b):
      pltpu.async_copy(
          out_bufs[b],
          out_hbm.at[pl.ds((base0 + t * KI) * EMBED_DIM, KI * EMBED_DIM)],
          wsems[b])

    def wait_wb(b):
      pltpu.make_async_copy(out_bufs[b],
                            out_hbm.at[pl.ds(0, KI * EMBED_DIM)],
                            wsems[b]).wait()

    def compute(b):
      ivb = idx_bufs[b]
      ovb = out_bufs[b]

      def group(g):
        iv = ivb[pl.ds(g * 16, 16)]
        a0 = iv * EMBED_DIM
        st0 = st_iota + g * (16 * EMBED_DIM)
        vals = [None] * EMBED_DIM

        def store(c):
          plsc.store_scatter(ovb, [st0 + c], vals[c])

        for c in range(EMBED_DIM):
          vals[c] = plsc.load_gather(tab_v, [a0 + c])
          if c >= LOOKAHEAD:
            store(c - LOOKAHEAD)
        for c in range(EMBED_DIM - LOOKAHEAD, EMBED_DIM):
          store(c)

      plsc.parallel_loop(0, GROUPS, 1, unroll=1)(group)

    issue_idx(0, 0)
    issue_idx(1, 1)

    def step2(tt, carry):
      for b in (0, 1):
        t = 2 * tt + b
        wait_idx(b)

        @pl.when(tt > 0)
        def _():
          wait_wb(b)

        compute(b)
        issue_wb(t, b)

        @pl.when(tt < STEPS // 2 - 1)
        def _():
          issue_idx(t + 2, b)
      return carry

    lax.fori_loop(0, STEPS // 2, step2, 0)
    wait_wb(0)
    wait_wb(1)

  return emb_kernel


_emb = _make_emb_kernel()


@jax.jit
def kernel(np_batch, table):
  flat_idx = np_batch.reshape(-1).astype(jnp.int32)
  flat_tab = table.reshape(-1)
  out = _emb(flat_idx, flat_tab)
  return out.reshape(BATCH, SEQ_LEN, WORD_LEN, EMBED_DIM)


# speedup vs baseline: 10.7947x; 1.4828x over previous
import functools

import jax
import jax.numpy as jnp
from jax import lax
from jax.experimental import pallas as pl
from jax.experimental.pallas import tpu as pltpu
from jax.experimental.pallas import tpu_sc as plsc

BATCH = 1024
SEQ_LEN = 200
WORD_LEN = 20
VOCAB_N = 1000
EMBED_DIM = 32
B_TOTAL = BATCH * SEQ_LEN * WORD_LEN

_INFO = plsc.get_sparse_core_info()
NC = _INFO.num_cores
NS = _INFO.num_subcores
NW = NC * NS
N_PER_W = B_TOTAL // NW

KI = 1024
IROWS = KI // 128
GROUPS = KI // 16
STEPS = N_PER_W // KI
LOOKAHEAD = 8


def _make_emb_kernel():
  mesh = plsc.VectorSubcoreMesh(core_axis_name="c", subcore_axis_name="s")

  @functools.partial(
      pl.kernel,
      mesh=mesh,
      compiler_params=pltpu.CompilerParams(needs_layout_passes=False),
      out_type=jax.ShapeDtypeStruct((B_TOTAL * EMBED_DIM,), jnp.float32),
      scratch_types=[
          pltpu.VMEM((VOCAB_N * EMBED_DIM,), jnp.float32),
          pltpu.VMEM((IROWS, 128), jnp.int32),
          pltpu.VMEM((IROWS, 128), jnp.int32),
          pltpu.VMEM((KI * EMBED_DIM,), jnp.float32),
          pltpu.VMEM((KI * EMBED_DIM,), jnp.float32),
          pltpu.SemaphoreType.DMA,
          pltpu.SemaphoreType.DMA,
          pltpu.SemaphoreType.DMA,
          pltpu.SemaphoreType.DMA,
          pltpu.SemaphoreType.DMA,
      ],
  )
  def emb_kernel(idx_hbm, tabf_hbm, out_hbm, tab_v, idx_v0, idx_v1,
                 out_v0, out_v1, tab_sem, isem0, isem1, wsem0, wsem1):
    wid = lax.axis_index("s") * NC + lax.axis_index("c")
    base0 = wid * N_PER_W
    irow0 = base0 // 128
    pltpu.async_copy(tabf_hbm, tab_v, tab_sem).wait()

    idx_bufs = (idx_v0, idx_v1)
    out_bufs = (out_v0, out_v1)
    isems = (isem0, isem1)
    wsems = (wsem0, wsem1)
    iota16 = lax.iota(jnp.int32, 16)
    st_iota = iota16 * EMBED_DIM

    def issue_idx(t, b):
      row = pl.multiple_of(irow0 + t * IROWS, IROWS)
      pltpu.async_copy(idx_hbm.at[pl.ds(row, IROWS)],
                       idx_bufs[b], isems[b])

    def wait_idx(b):
      pltpu.make_async_copy(idx_hbm.at[pl.ds(0, IROWS)],
                            idx_bufs[b], isems[b]).wait()

    def issue_wb(t, b):
      pltpu.async_copy(
          out_bufs[b],
          out_hbm.at[pl.ds((base0 + t * KI) * EMBED_DIM, KI * EMBED_DIM)],
          wsems[b])

    def wait_wb(b):
      pltpu.make_async_copy(out_bufs[b],
                            out_hbm.at[pl.ds(0, KI * EMBED_DIM)],
                            wsems[b]).wait()

    def compute(b):
      ivb = idx_bufs[b]
      ovb = out_bufs[b]

      def group(g):
        iv = ivb[g >> 3, pl.ds((g & 7) * 16, 16)]
        a0 = iv * EMBED_DIM
        st0 = st_iota + g * (16 * EMBED_DIM)
        vals = [None] * EMBED_DIM
        rots = [None] * EMBED_DIM

        def store(r):
          plsc.store_scatter(ovb, [st0 + rots[r]], vals[r])

        for r in range(EMBED_DIM):
          rots[r] = (iota16 + r) & (EMBED_DIM - 1)
          vals[r] = plsc.load_gather(tab_v, [a0 + rots[r]])
          if r >= LOOKAHEAD:
            store(r - LOOKAHEAD)
        for r in range(EMBED_DIM - LOOKAHEAD, EMBED_DIM):
          store(r)

      plsc.parallel_loop(0, GROUPS, 1, unroll=1)(group)

    issue_idx(0, 0)
    issue_idx(1, 1)

    def step2(tt, carry):
      for b in (0, 1):
        t = 2 * tt + b
        wait_idx(b)

        @pl.when(tt > 0)
        def _():
          wait_wb(b)

        compute(b)
        issue_wb(t, b)

        @pl.when(t + 2 < STEPS)
        def _():
          issue_idx(t + 2, b)
      return carry

    lax.fori_loop(0, STEPS // 2, step2, 0)
    wait_idx(0)
    wait_wb(0)
    compute(0)
    issue_wb(STEPS - 1, 0)
    wait_wb(0)
    wait_wb(1)

  return emb_kernel


_emb = _make_emb_kernel()


@jax.jit
def kernel(np_batch, table):
  idx2d = np_batch.reshape(B_TOTAL // 128, 128).astype(jnp.int32)
  flat_tab = table.reshape(-1)
  out = _emb(idx2d, flat_tab)
  return out.reshape(BATCH, SEQ_LEN, WORD_LEN, EMBED_DIM)
